# baseline (speedup 1.0000x reference)
import functools

import jax
import jax.numpy as jnp
from jax.experimental import pallas as pl


def _mm_kernel(x_ref, w_ref, b_ref, o_ref):
    x = x_ref[0]
    acc = jax.lax.dot_general(
        x, w_ref[...], (((0,), (0,)), ((), ())),
        preferred_element_type=jnp.float32,
    )
    o_ref[0] = acc + b_ref[...]


@functools.partial(jax.jit, static_argnames=("hw_tile",))
def _head_scale(feat, W, b, hw_tile):
    Bn, C, Sy, Sx = feat.shape
    HW = Sy * Sx
    O = W.shape[0]
    x = feat.reshape(Bn, C, HW)
    wt = W.T
    bias = b.reshape(1, O)
    out = pl.pallas_call(
        _mm_kernel,
        grid=(Bn, HW // hw_tile),
        in_specs=[
            pl.BlockSpec((1, C, hw_tile), lambda i, j: (i, 0, j)),
            pl.BlockSpec((C, O), lambda i, j: (0, 0)),
            pl.BlockSpec((1, O), lambda i, j: (0, 0)),
        ],
        out_specs=pl.BlockSpec((1, hw_tile, O), lambda i, j: (i, j, 0)),
        out_shape=jax.ShapeDtypeStruct((Bn, HW, O), jnp.float32),
    )(x, wt, bias)
    return out.reshape(Bn, Sy, Sx, O)


def kernel(feat0, feat1, feat2, W0, b0, W1, b1, W2, b2):
    out0 = _head_scale(feat0, W0, b0, hw_tile=512)
    out1 = _head_scale(feat1, W1, b1, hw_tile=512)
    out2 = _head_scale(feat2, W2, b2, hw_tile=256)
    return (out0, out1, out2)


# speedup vs baseline: 1.3290x; 1.3290x over previous
import functools

import jax
import jax.numpy as jnp
from jax.experimental import pallas as pl


def _mm_kernel(x_ref, w_ref, b_ref, o_ref):
    x = x_ref[0]
    acc = jax.lax.dot_general(
        x, w_ref[...], (((0,), (0,)), ((), ())),
        preferred_element_type=jnp.float32,
    )
    o_ref[0] = acc + b_ref[...]


@jax.jit
def _head_scale(feat, W, b):
    Bn, C, Sy, Sx = feat.shape
    HW = Sy * Sx
    O = W.shape[0]
    x = feat.reshape(Bn, C, HW)
    wt = W.T
    bias = b.reshape(1, O)
    out = pl.pallas_call(
        _mm_kernel,
        grid=(Bn,),
        in_specs=[
            pl.BlockSpec((1, C, HW), lambda i: (i, 0, 0)),
            pl.BlockSpec((C, O), lambda i: (0, 0)),
            pl.BlockSpec((1, O), lambda i: (0, 0)),
        ],
        out_specs=pl.BlockSpec((1, HW, O), lambda i: (i, 0, 0)),
        out_shape=jax.ShapeDtypeStruct((Bn, HW, O), jnp.float32),
    )(x, wt, bias)
    return out.reshape(Bn, Sy, Sx, O)


def kernel(feat0, feat1, feat2, W0, b0, W1, b1, W2, b2):
    out0 = _head_scale(feat0, W0, b0)
    out1 = _head_scale(feat1, W1, b1)
    out2 = _head_scale(feat2, W2, b2)
    return (out0, out1, out2)


# speedup vs baseline: 1.4060x; 1.0579x over previous
import jax
import jax.numpy as jnp
from jax.experimental import pallas as pl


def _fused_kernel(x0_ref, x1_ref, x2_ref, w0_ref, w1_ref, w2_ref, b_ref,
                  o0_ref, o1_ref, o2_ref):
    dims = (((0,), (0,)), ((), ()))
    o0_ref[0] = jax.lax.dot_general(
        x0_ref[0], w0_ref[...], dims, preferred_element_type=jnp.float32
    ) + b_ref[0:1, :]
    o1_ref[0] = jax.lax.dot_general(
        x1_ref[0], w1_ref[...], dims, preferred_element_type=jnp.float32
    ) + b_ref[1:2, :]
    o2_ref[0] = jax.lax.dot_general(
        x2_ref[0], w2_ref[...], dims, preferred_element_type=jnp.float32
    ) + b_ref[2:3, :]


@jax.jit
def _head(feat0, feat1, feat2, W0, b0, W1, b1, W2, b2):
    Bn = feat0.shape[0]
    shapes = [feat0.shape, feat1.shape, feat2.shape]
    hws = [sy * sx for (_, _, sy, sx) in shapes]
    cs = [c for (_, c, _, _) in shapes]
    O = W0.shape[0]

    x0 = feat0.reshape(Bn, cs[0], hws[0])
    x1 = feat1.reshape(Bn, cs[1], hws[1])
    x2 = feat2.reshape(Bn, cs[2], hws[2])
    wt0, wt1, wt2 = W0.T, W1.T, W2.T
    bias = jnp.stack([b0, b1, b2])

    o0, o1, o2 = pl.pallas_call(
        _fused_kernel,
        grid=(Bn,),
        in_specs=[
            pl.BlockSpec((1, cs[0], hws[0]), lambda i: (i, 0, 0)),
            pl.BlockSpec((1, cs[1], hws[1]), lambda i: (i, 0, 0)),
            pl.BlockSpec((1, cs[2], hws[2]), lambda i: (i, 0, 0)),
            pl.BlockSpec((cs[0], O), lambda i: (0, 0)),
            pl.BlockSpec((cs[1], O), lambda i: (0, 0)),
            pl.BlockSpec((cs[2], O), lambda i: (0, 0)),
            pl.BlockSpec((3, O), lambda i: (0, 0)),
        ],
        out_specs=[
            pl.BlockSpec((1, hws[0], O), lambda i: (i, 0, 0)),
            pl.BlockSpec((1, hws[1], O), lambda i: (i, 0, 0)),
            pl.BlockSpec((1, hws[2], O), lambda i: (i, 0, 0)),
        ],
        out_shape=[
            jax.ShapeDtypeStruct((Bn, hws[0], O), jnp.float32),
            jax.ShapeDtypeStruct((Bn, hws[1], O), jnp.float32),
            jax.ShapeDtypeStruct((Bn, hws[2], O), jnp.float32),
        ],
    )(x0, x1, x2, wt0, wt1, wt2, bias)

    outs = []
    for o, (_, _, sy, sx) in zip((o0, o1, o2), shapes):
        outs.append(o.reshape(Bn, sy, sx, O))
    return tuple(outs)


def kernel(feat0, feat1, feat2, W0, b0, W1, b1, W2, b2):
    return _head(feat0, feat1, feat2, W0, b0, W1, b1, W2, b2)


# speedup vs baseline: 3.3434x; 2.3779x over previous
import jax
import jax.numpy as jnp
from jax.experimental import pallas as pl
from jax.experimental.pallas import tpu as pltpu


def _fused_kernel(x0_ref, x1_ref, x2_ref, w0_ref, w1_ref, w2_ref, b_ref,
                  o0_ref, o1_ref, o2_ref):
    dims = (((1,), (1,)), ((), ()))
    o0_ref[0] = jax.lax.dot_general(
        x0_ref[0].astype(jnp.bfloat16), w0_ref[...], dims,
        preferred_element_type=jnp.float32,
    ) + b_ref[0:1, :]
    o1_ref[0] = jax.lax.dot_general(
        x1_ref[0].astype(jnp.bfloat16), w1_ref[...], dims,
        preferred_element_type=jnp.float32,
    ) + b_ref[1:2, :]
    o2_ref[0] = jax.lax.dot_general(
        x2_ref[0].astype(jnp.bfloat16), w2_ref[...], dims,
        preferred_element_type=jnp.float32,
    ) + b_ref[2:3, :]


@jax.jit
def _head(feat0, feat1, feat2, W0, b0, W1, b1, W2, b2):
    Bn = feat0.shape[0]
    shapes = [feat0.shape, feat1.shape, feat2.shape]
    O = W0.shape[0]

    xs = [
        f.transpose(0, 2, 3, 1).reshape(Bn, sy * sx, c)
        for f, (_, c, sy, sx) in zip((feat0, feat1, feat2), shapes)
    ]
    ws = [W.astype(jnp.bfloat16) for W in (W0, W1, W2)]
    bias = jnp.stack([b0, b1, b2])

    in_specs = (
        [pl.BlockSpec((1, sy * sx, c), lambda i: (i, 0, 0))
         for (_, c, sy, sx) in shapes]
        + [pl.BlockSpec((O, c), lambda i: (0, 0)) for (_, c, _, _) in shapes]
        + [pl.BlockSpec((3, O), lambda i: (0, 0))]
    )

    o0, o1, o2 = pl.pallas_call(
        _fused_kernel,
        grid=(Bn,),
        in_specs=in_specs,
        out_specs=[
            pl.BlockSpec((1, sy * sx, O), lambda i: (i, 0, 0))
            for (_, _, sy, sx) in shapes
        ],
        out_shape=[
            jax.ShapeDtypeStruct((Bn, sy * sx, O), jnp.float32)
            for (_, _, sy, sx) in shapes
        ],
        compiler_params=pltpu.CompilerParams(
            vmem_limit_bytes=100 * 1024 * 1024,
        ),
    )(*xs, *ws, bias)

    outs = []
    for o, (_, _, sy, sx) in zip((o0, o1, o2), shapes):
        outs.append(o.reshape(Bn, sy, sx, O))
    return tuple(outs)


def kernel(feat0, feat1, feat2, W0, b0, W1, b1, W2, b2):
    return _head(feat0, feat1, feat2, W0, b0, W1, b1, W2, b2)


# speedup vs baseline: 3.6524x; 1.0924x over previous
import jax
import jax.numpy as jnp
from jax.experimental import pallas as pl
from jax.experimental.pallas import tpu as pltpu


def _fused_kernel(x0_ref, x1_ref, x2_ref, w0_ref, w1_ref, w2_ref,
                  b0_ref, b1_ref, b2_ref, o0_ref, o1_ref, o2_ref):
    dims = (((1,), (1,)), ((), ()))
    for x_ref, w_ref, b_ref, o_ref in (
        (x0_ref, w0_ref, b0_ref, o0_ref),
        (x1_ref, w1_ref, b1_ref, o1_ref),
        (x2_ref, w2_ref, b2_ref, o2_ref),
    ):
        o_ref[0] = jax.lax.dot_general(
            x_ref[0].astype(jnp.bfloat16),
            w_ref[...].astype(jnp.bfloat16),
            dims,
            preferred_element_type=jnp.float32,
        ) + b_ref[...]


@jax.jit
def _head(feat0, feat1, feat2, W0, b0, W1, b1, W2, b2):
    Bn = feat0.shape[0]
    shapes = [feat0.shape, feat1.shape, feat2.shape]
    O = W0.shape[0]
    SPLIT = 2

    xs = [
        f.transpose(0, 2, 3, 1).reshape(Bn, sy * sx, c)
        for f, (_, c, sy, sx) in zip((feat0, feat1, feat2), shapes)
    ]
    bs = [b.reshape(1, O) for b in (b0, b1, b2)]

    def x_spec(c, hw):
        return pl.BlockSpec((1, hw // SPLIT, c), lambda i, j: (i, j, 0))

    def w_spec(c):
        return pl.BlockSpec((O, c), lambda i, j: (0, 0))

    b_spec = pl.BlockSpec((1, O), lambda i, j: (0, 0))

    o0, o1, o2 = pl.pallas_call(
        _fused_kernel,
        grid=(Bn, SPLIT),
        in_specs=(
            [x_spec(c, sy * sx) for (_, c, sy, sx) in shapes]
            + [w_spec(c) for (_, c, _, _) in shapes]
            + [b_spec, b_spec, b_spec]
        ),
        out_specs=[
            pl.BlockSpec((1, sy * sx // SPLIT, O), lambda i, j: (i, j, 0))
            for (_, _, sy, sx) in shapes
        ],
        out_shape=[
            jax.ShapeDtypeStruct((Bn, sy * sx, O), jnp.float32)
            for (_, _, sy, sx) in shapes
        ],
        compiler_params=pltpu.CompilerParams(
            vmem_limit_bytes=100 * 1024 * 1024,
        ),
    )(*xs, W0, W1, W2, *bs)

    outs = []
    for o, (_, _, sy, sx) in zip((o0, o1, o2), shapes):
        outs.append(o.reshape(Bn, sy, sx, O))
    return tuple(outs)


def kernel(feat0, feat1, feat2, W0, b0, W1, b1, W2, b2):
    return _head(feat0, feat1, feat2, W0, b0, W1, b1, W2, b2)


# speedup vs baseline: 4.0489x; 1.1085x over previous
import jax
import jax.numpy as jnp
from jax.experimental import pallas as pl
from jax.experimental.pallas import tpu as pltpu


def _fused_kernel(x0_ref, x1_ref, x2_ref, w0_ref, w1_ref, w2_ref,
                  b0_ref, b1_ref, b2_ref, o0_ref, o1_ref, o2_ref,
                  w0s_ref, w1s_ref, w2s_ref):
    @pl.when(pl.program_id(0) == 0)
    def _cast_weights():
        w0s_ref[...] = w0_ref[...].astype(jnp.bfloat16)
        w1s_ref[...] = w1_ref[...].astype(jnp.bfloat16)
        w2s_ref[...] = w2_ref[...].astype(jnp.bfloat16)

    dims = (((1,), (1,)), ((), ()))
    for x_ref, ws_ref, b_ref, o_ref in (
        (x0_ref, w0s_ref, b0_ref, o0_ref),
        (x1_ref, w1s_ref, b1_ref, o1_ref),
        (x2_ref, w2s_ref, b2_ref, o2_ref),
    ):
        o_ref[0] = jax.lax.dot_general(
            x_ref[0].astype(jnp.bfloat16), ws_ref[...], dims,
            preferred_element_type=jnp.float32,
        ) + b_ref[...]


@jax.jit
def _head(feat0, feat1, feat2, W0, b0, W1, b1, W2, b2):
    Bn = feat0.shape[0]
    shapes = [feat0.shape, feat1.shape, feat2.shape]
    O = W0.shape[0]

    xs = [
        f.transpose(0, 2, 3, 1).reshape(Bn, sy * sx, c)
        for f, (_, c, sy, sx) in zip((feat0, feat1, feat2), shapes)
    ]
    bs = [b.reshape(1, O) for b in (b0, b1, b2)]

    o0, o1, o2 = pl.pallas_call(
        _fused_kernel,
        grid=(Bn,),
        in_specs=(
            [pl.BlockSpec((1, sy * sx, c), lambda i: (i, 0, 0))
             for (_, c, sy, sx) in shapes]
            + [pl.BlockSpec((O, c), lambda i: (0, 0))
               for (_, c, _, _) in shapes]
            + [pl.BlockSpec((1, O), lambda i: (0, 0))] * 3
        ),
        out_specs=[
            pl.BlockSpec((1, sy * sx, O), lambda i: (i, 0, 0))
            for (_, _, sy, sx) in shapes
        ],
        out_shape=[
            jax.ShapeDtypeStruct((Bn, sy * sx, O), jnp.float32)
            for (_, _, sy, sx) in shapes
        ],
        scratch_shapes=[
            pltpu.VMEM((O, c), jnp.bfloat16) for (_, c, _, _) in shapes
        ],
        compiler_params=pltpu.CompilerParams(
            vmem_limit_bytes=100 * 1024 * 1024,
        ),
    )(*xs, W0, W1, W2, *bs)

    outs = []
    for o, (_, _, sy, sx) in zip((o0, o1, o2), shapes):
        outs.append(o.reshape(Bn, sy, sx, O))
    return tuple(outs)


def kernel(feat0, feat1, feat2, W0, b0, W1, b1, W2, b2):
    return _head(feat0, feat1, feat2, W0, b0, W1, b1, W2, b2)


# speedup vs baseline: 4.0570x; 1.0020x over previous
import jax
import jax.numpy as jnp
from jax.experimental import pallas as pl
from jax.experimental.pallas import tpu as pltpu


def _fused_kernel(x0_ref, x1_ref, x2_ref, w0_ref, w1_ref, w2_ref,
                  b0_ref, b1_ref, b2_ref, o0_ref, o1_ref, o2_ref,
                  w0s_ref, w1s_ref, w2s_ref):
    w0s_ref[...] = w0_ref[...].astype(jnp.bfloat16)
    w1s_ref[...] = w1_ref[...].astype(jnp.bfloat16)
    w2s_ref[...] = w2_ref[...].astype(jnp.bfloat16)

    dims = (((1,), (1,)), ((), ()))
    for x_ref, ws_ref, b_ref, o_ref in (
        (x0_ref, w0s_ref, b0_ref, o0_ref),
        (x1_ref, w1s_ref, b1_ref, o1_ref),
        (x2_ref, w2s_ref, b2_ref, o2_ref),
    ):
        o_ref[0] = jax.lax.dot_general(
            x_ref[0].astype(jnp.bfloat16), ws_ref[...], dims,
            preferred_element_type=jnp.float32,
        ) + b_ref[...]


@jax.jit
def _head(feat0, feat1, feat2, W0, b0, W1, b1, W2, b2):
    Bn = feat0.shape[0]
    shapes = [feat0.shape, feat1.shape, feat2.shape]
    O = W0.shape[0]

    xs = [
        f.transpose(0, 2, 3, 1).reshape(Bn, sy * sx, c)
        for f, (_, c, sy, sx) in zip((feat0, feat1, feat2), shapes)
    ]
    bs = [b.reshape(1, O) for b in (b0, b1, b2)]

    o0, o1, o2 = pl.pallas_call(
        _fused_kernel,
        grid=(Bn,),
        in_specs=(
            [pl.BlockSpec((1, sy * sx, c), lambda i: (i, 0, 0))
             for (_, c, sy, sx) in shapes]
            + [pl.BlockSpec((O, c), lambda i: (0, 0))
               for (_, c, _, _) in shapes]
            + [pl.BlockSpec((1, O), lambda i: (0, 0))] * 3
        ),
        out_specs=[
            pl.BlockSpec((1, sy * sx, O), lambda i: (i, 0, 0))
            for (_, _, sy, sx) in shapes
        ],
        out_shape=[
            jax.ShapeDtypeStruct((Bn, sy * sx, O), jnp.float32)
            for (_, _, sy, sx) in shapes
        ],
        scratch_shapes=[
            pltpu.VMEM((O, c), jnp.bfloat16) for (_, c, _, _) in shapes
        ],
        compiler_params=pltpu.CompilerParams(
            vmem_limit_bytes=100 * 1024 * 1024,
            dimension_semantics=("parallel",),
        ),
    )(*xs, W0, W1, W2, *bs)

    outs = []
    for o, (_, _, sy, sx) in zip((o0, o1, o2), shapes):
        outs.append(o.reshape(Bn, sy, sx, O))
    return tuple(outs)


def kernel(feat0, feat1, feat2, W0, b0, W1, b1, W2, b2):
    return _head(feat0, feat1, feat2, W0, b0, W1, b1, W2, b2)
